# Initial kernel scaffold; baseline (speedup 1.0000x reference)
import jax
import jax.numpy as jnp
from jax.experimental import pallas as pl


def kernel(x, coeffs, bias, knots):
    raise NotImplementedError("write your pallas kernel here")


# speedup vs baseline: 116.0102x; 116.0102x over previous
import jax
import jax.numpy as jnp
from jax.experimental import pallas as pl

_B, _I, _O, _K = 2048, 64, 64, 64
_BT = 256


def _slopes_body(coeffs_ref, knots_ref, out_ref):
    y = coeffs_ref[...]
    kn = knots_ref[...]
    h = kn[:, 1:] - kn[:, :-1]
    hb = h[None]
    delta = (y[..., 1:] - y[..., :-1]) / (hb + 1e-12)
    hkm1 = hb[..., :-1]
    hk = hb[..., 1:]
    w1 = 2.0 * hk + hkm1
    w2 = hk + 2.0 * hkm1
    same_sign = delta[..., :-1] * delta[..., 1:] > 0
    d_int = (w1 + w2) / (w1 / (delta[..., :-1] + 1e-12)
                         + w2 / (delta[..., 1:] + 1e-12))
    d_interior = jnp.where(same_sign, d_int, jnp.zeros_like(d_int))

    h0 = h[:, 0:1][None]
    h1 = h[:, 1:2][None]
    hm1 = h[:, -1:][None]
    hm2 = h[:, -2:-1][None]
    dl0 = delta[..., 0:1]
    dl1 = delta[..., 1:2]
    dlm1 = delta[..., -1:]
    dlm2 = delta[..., -2:-1]
    d0 = ((2.0 * h0 + h1) * dl0 - h0 * dl1) / (h0 + h1 + 1e-12)
    dN = ((2.0 * hm1 + hm2) * dlm1 - hm1 * dlm2) / (hm1 + hm2 + 1e-12)

    def limit(di, deltai):
        di = jnp.where(di * deltai <= 0, jnp.zeros_like(di), di)
        di = jnp.where(jnp.abs(di) > 3.0 * jnp.abs(deltai), 3.0 * deltai, di)
        return di

    d0 = limit(d0, dl0)
    dN = limit(dN, dlm1)
    d = jnp.concatenate([d0, d_interior, dN], axis=-1)
    hl = jnp.concatenate([h, h[:, -1:]], axis=1)[None]
    out_ref[...] = d * hl


def _spline_body(x_ref, c_ref, hd_ref, bias_ref, out_ref):
    x = x_ref[...]
    u = jnp.clip(x, 0.0, 1.0) * (_K - 1.0)
    kk = jax.lax.broadcasted_iota(jnp.int32, (_BT, _I, _K), 2).astype(jnp.float32)
    s = u[:, :, None] - kk
    r = jnp.minimum(jnp.abs(s), 1.0)
    q = (1.0 - r) * (1.0 - r)
    wc = (q * (1.0 + 2.0 * r)).reshape(_BT, _I * _K)
    wd = (q * s).reshape(_BT, _I * _K)
    dn = (((1,), (1,)), ((), ()))
    acc = jax.lax.dot_general(wc, c_ref[...], dn,
                              preferred_element_type=jnp.float32)
    acc = acc + jax.lax.dot_general(wd, hd_ref[...], dn,
                                    preferred_element_type=jnp.float32)
    out_ref[...] = acc + bias_ref[...]


def kernel(x, coeffs, bias, knots):
    hd = pl.pallas_call(
        _slopes_body,
        out_shape=jax.ShapeDtypeStruct((_O, _I, _K), jnp.float32),
    )(coeffs, knots.reshape(1, _K))

    c2 = coeffs.reshape(_O, _I * _K)
    hd2 = hd.reshape(_O, _I * _K)
    grid = _B // _BT
    out = pl.pallas_call(
        _spline_body,
        grid=(grid,),
        in_specs=[
            pl.BlockSpec((_BT, _I), lambda b: (b, 0)),
            pl.BlockSpec((_O, _I * _K), lambda b: (0, 0)),
            pl.BlockSpec((_O, _I * _K), lambda b: (0, 0)),
            pl.BlockSpec((1, _O), lambda b: (0, 0)),
        ],
        out_specs=pl.BlockSpec((_BT, _O), lambda b: (b, 0)),
        out_shape=jax.ShapeDtypeStruct((_B, _O), jnp.float32),
    )(x, c2, hd2, bias.reshape(1, _O))
    return out


# speedup vs baseline: 227.4049x; 1.9602x over previous
import jax
import jax.numpy as jnp
from jax.experimental import pallas as pl

_B, _I, _O, _K = 2048, 64, 64, 64
_BT = 256


def _slopes_body(coeffs_ref, knots_ref, out_ref):
    y = coeffs_ref[...]
    kn = knots_ref[...]
    h = kn[:, 1:] - kn[:, :-1]
    hb = h[None]
    delta = (y[..., 1:] - y[..., :-1]) / (hb + 1e-12)
    hkm1 = hb[..., :-1]
    hk = hb[..., 1:]
    w1 = 2.0 * hk + hkm1
    w2 = hk + 2.0 * hkm1
    same_sign = delta[..., :-1] * delta[..., 1:] > 0
    d_int = (w1 + w2) / (w1 / (delta[..., :-1] + 1e-12)
                         + w2 / (delta[..., 1:] + 1e-12))
    d_interior = jnp.where(same_sign, d_int, jnp.zeros_like(d_int))

    h0 = h[:, 0:1][None]
    h1 = h[:, 1:2][None]
    hm1 = h[:, -1:][None]
    hm2 = h[:, -2:-1][None]
    dl0 = delta[..., 0:1]
    dl1 = delta[..., 1:2]
    dlm1 = delta[..., -1:]
    dlm2 = delta[..., -2:-1]
    d0 = ((2.0 * h0 + h1) * dl0 - h0 * dl1) / (h0 + h1 + 1e-12)
    dN = ((2.0 * hm1 + hm2) * dlm1 - hm1 * dlm2) / (hm1 + hm2 + 1e-12)

    def limit(di, deltai):
        di = jnp.where(di * deltai <= 0, jnp.zeros_like(di), di)
        di = jnp.where(jnp.abs(di) > 3.0 * jnp.abs(deltai), 3.0 * deltai, di)
        return di

    d0 = limit(d0, dl0)
    dN = limit(dN, dlm1)
    d = jnp.concatenate([d0, d_interior, dN], axis=-1)
    hl = jnp.concatenate([h, h[:, -1:]], axis=1)[None]
    out_ref[...] = d * hl


def _spline_body(x_ref, c_ref, hd_ref, bias_ref, out_ref):
    n = _I * _K
    x = x_ref[...]
    u = jnp.clip(x, 0.0, 1.0) * (_K - 1.0)
    col = jax.lax.broadcasted_iota(jnp.int32, (_I, n), 1)
    row = jax.lax.broadcasted_iota(jnp.int32, (_I, n), 0)
    rep = ((col >> 6) == row).astype(jnp.float32)
    dn0 = (((1,), (0,)), ((), ()))
    urep = jax.lax.dot_general(u, rep, dn0,
                               preferred_element_type=jnp.float32)
    kk = (jax.lax.broadcasted_iota(jnp.int32, (_BT, n), 1)
          & (_K - 1)).astype(jnp.float32)
    s = urep - kk
    r = jnp.minimum(jnp.abs(s), 1.0)
    q = (1.0 - r) * (1.0 - r)
    wc = q * (1.0 + 2.0 * r)
    wd = q * s
    dn = (((1,), (1,)), ((), ()))
    acc = jax.lax.dot_general(wc, c_ref[...], dn,
                              preferred_element_type=jnp.float32)
    acc = acc + jax.lax.dot_general(wd, hd_ref[...], dn,
                                    preferred_element_type=jnp.float32)
    out_ref[...] = acc + bias_ref[...]


def kernel(x, coeffs, bias, knots):
    hd = pl.pallas_call(
        _slopes_body,
        out_shape=jax.ShapeDtypeStruct((_O, _I, _K), jnp.float32),
    )(coeffs, knots.reshape(1, _K))

    c2 = coeffs.reshape(_O, _I * _K)
    hd2 = hd.reshape(_O, _I * _K)
    grid = _B // _BT
    out = pl.pallas_call(
        _spline_body,
        grid=(grid,),
        in_specs=[
            pl.BlockSpec((_BT, _I), lambda b: (b, 0)),
            pl.BlockSpec((_O, _I * _K), lambda b: (0, 0)),
            pl.BlockSpec((_O, _I * _K), lambda b: (0, 0)),
            pl.BlockSpec((1, _O), lambda b: (0, 0)),
        ],
        out_specs=pl.BlockSpec((_BT, _O), lambda b: (b, 0)),
        out_shape=jax.ShapeDtypeStruct((_B, _O), jnp.float32),
    )(x, c2, hd2, bias.reshape(1, _O))
    return out
